# baseline (speedup 1.0000x reference)
import jax
import jax.numpy as jnp
from jax import lax
from jax.experimental import pallas as pl
from jax.experimental.pallas import tpu as pltpu
from jax.experimental.pallas import tpu_sc as plsc

_B = 4
_L = 8192
_D = 1024
_NC = 2
_NS = 16
_NW = _NC * _NS
_BG = 2
_NSLICE = _NW // (_B // _BG)
_RPW = _L // _NSLICE
_C = 8
_NCH = _RPW // _C
_NSLOT = 5


def _sc_body(x_hbm, e_hbm, o_hbm, xb, eb, *sems):
    sin = sems[:_NSLOT]
    sout = sems[_NSLOT:]
    wid = lax.axis_index("s") * _NC + lax.axis_index("c")
    bg = lax.rem(wid, _B // _BG) * _BG
    base = lax.div(wid, _B // _BG) * _RPW

    def rows(c):
        return pl.ds(pl.multiple_of(base + c * _C, _C), _C)

    def fire_in(s, c):
        r = rows(c)
        pltpu.async_copy(e_hbm.at[r], eb.at[s], sin[s])
        for b in range(_BG):
            pltpu.async_copy(x_hbm.at[bg + b, r], xb.at[s, b], sin[s])

    def wait_in(s, c):
        r = rows(c)
        pltpu.make_async_copy(e_hbm.at[r], eb.at[s], sin[s]).wait()
        for b in range(_BG):
            pltpu.make_async_copy(x_hbm.at[bg + b, r], xb.at[s, b], sin[s]).wait()

    def fire_out(s, c):
        r = rows(c)
        for b in range(_BG):
            pltpu.async_copy(xb.at[s, b], o_hbm.at[bg + b, r], sout[s])

    def wait_out(s, c):
        r = rows(c)
        for b in range(_BG):
            pltpu.make_async_copy(xb.at[s, b], o_hbm.at[bg + b, r], sout[s]).wait()

    def compute(s):
        er = eb.at[s]
        xr = [xb.at[s, b] for b in range(_BG)]

        @plsc.parallel_loop(0, _D, step=16, unroll=4)
        def step(i):
            sl = pl.ds(i, 16)
            for r in range(_C):
                v = er[r, sl]
                for b in range(_BG):
                    xr[b][r, sl] = xr[b][r, sl] + v

    for s in range(_NSLOT):
        fire_in(s, s)

    def body(k, carry):
        for j in range(_NSLOT):
            c = _NSLOT * k + j

            @pl.when(c < _NCH)
            def _():
                wait_in(j, c)
                compute(j)
                fire_out(j, c)
                p = c - 3
                sp = (j + 2) % _NSLOT

                @pl.when((p >= 0) & (p + _NSLOT < _NCH))
                def _():
                    wait_out(sp, p)
                    fire_in(sp, p + _NSLOT)

        return carry

    lax.fori_loop(0, (_NCH + _NSLOT - 1) // _NSLOT, body, 0)

    for c in range(_NCH - _NSLOT, _NCH):
        wait_out(c % _NSLOT, c)


def kernel(input, emb):
    run = pl.kernel(
        _sc_body,
        out_type=jax.ShapeDtypeStruct((_B, _L, _D), jnp.float32),
        mesh=plsc.VectorSubcoreMesh(core_axis_name="c", subcore_axis_name="s"),
        compiler_params=pltpu.CompilerParams(use_tc_tiling_on_sc=True),
        scratch_types=[
            pltpu.VMEM((_NSLOT, _BG, _C, _D), jnp.float32),
            pltpu.VMEM((_NSLOT, _C, _D), jnp.float32),
        ]
        + [pltpu.SemaphoreType.DMA] * (2 * _NSLOT),
    )
    return run(input, emb)


# speedup vs baseline: 1.1146x; 1.1146x over previous
import jax
import jax.numpy as jnp
from jax import lax
from jax.experimental import pallas as pl
from jax.experimental.pallas import tpu as pltpu
from jax.experimental.pallas import tpu_sc as plsc

_B = 4
_L = 8192
_D = 1024
_NC = 2
_NS = 16
_NW = _NC * _NS
_RPW = _L // _NW
_C = 8
_NCH = _RPW // _C
_NSLOT = 3


def _sc_body(x_hbm, e_hbm, o_hbm, xb, eb, *sems):
    sin = sems[:_NSLOT]
    sout = sems[_NSLOT:]
    wid = lax.axis_index("s") * _NC + lax.axis_index("c")
    base = wid * _RPW

    def rows(c):
        return pl.ds(pl.multiple_of(base + c * _C, _C), _C)

    def fire_in(s, c):
        r = rows(c)
        pltpu.async_copy(e_hbm.at[r], eb.at[s], sin[s])
        pltpu.async_copy(x_hbm.at[:, r], xb.at[s], sin[s])

    def wait_in(s, c):
        r = rows(c)
        pltpu.make_async_copy(e_hbm.at[r], eb.at[s], sin[s]).wait()
        pltpu.make_async_copy(x_hbm.at[:, r], xb.at[s], sin[s]).wait()

    def fire_out(s, c):
        r = rows(c)
        pltpu.async_copy(xb.at[s], o_hbm.at[:, r], sout[s])

    def wait_out(s, c):
        r = rows(c)
        pltpu.make_async_copy(xb.at[s], o_hbm.at[:, r], sout[s]).wait()

    def compute(s):
        er = eb.at[s]
        xr = [xb.at[s, b] for b in range(_B)]

        @plsc.parallel_loop(0, _D, step=16, unroll=4)
        def step(i):
            sl = pl.ds(i, 16)
            for r in range(_C):
                v = er[r, sl]
                for b in range(_B):
                    xr[b][r, sl] = xr[b][r, sl] + v

    for s in range(_NSLOT):
        fire_in(s, s)

    def body(k, carry):
        for j in range(_NSLOT):
            c = _NSLOT * k + j

            @pl.when(c < _NCH)
            def _():
                wait_in(j, c)
                compute(j)
                fire_out(j, c)
                p = c - 1
                sp = (j + 2) % _NSLOT

                @pl.when((p >= 0) & (p + _NSLOT < _NCH))
                def _():
                    wait_out(sp, p)
                    fire_in(sp, p + _NSLOT)

        return carry

    lax.fori_loop(0, (_NCH + _NSLOT - 1) // _NSLOT, body, 0)

    for c in range(_NCH - _NSLOT, _NCH):
        wait_out(c % _NSLOT, c)


def kernel(input, emb):
    run = pl.kernel(
        _sc_body,
        out_type=jax.ShapeDtypeStruct((_B, _L, _D), jnp.float32),
        mesh=plsc.VectorSubcoreMesh(core_axis_name="c", subcore_axis_name="s"),
        compiler_params=pltpu.CompilerParams(use_tc_tiling_on_sc=True),
        scratch_types=[
            pltpu.VMEM((_NSLOT, _B, _C, _D), jnp.float32),
            pltpu.VMEM((_NSLOT, _C, _D), jnp.float32),
        ]
        + [pltpu.SemaphoreType.DMA] * (2 * _NSLOT),
    )
    return run(input, emb)
